# baseline (speedup 1.0000x reference)
import jax
import jax.numpy as jnp
from jax.experimental import pallas as pl
from jax.experimental.pallas import tpu as pltpu


def _actor_t_kernel(x_ref, w1_ref, b1_ref, w2_ref, b2_ref, w3_ref, b3_ref,
                    s_ref, means_ref, stds_ref):
    x = x_ref[...]
    h1 = jnp.dot(w1_ref[...], x, preferred_element_type=jnp.float32)
    h1 = jnp.maximum(h1 + b1_ref[...], 0.0)
    h2 = jnp.dot(w2_ref[...], h1, preferred_element_type=jnp.float32)
    h2 = jnp.maximum(h2 + b2_ref[...], 0.0)
    m = jnp.dot(w3_ref[...], h2, preferred_element_type=jnp.float32)
    means_ref[...] = (m + b3_ref[...]).astype(means_ref.dtype)
    stds_ref[...] = jnp.broadcast_to(s_ref[...], stds_ref.shape)


def kernel(x, w1, b1, w2, b2, w3, b3, logstds):
    batch, state_size = x.shape
    action_size = w3.shape[1]

    xt = x.T
    w1t, w2t, w3t = w1.T, w2.T, w3.T
    b1t, b2t, b3t = b1.T, b2.T, b3.T
    st = jnp.minimum(jnp.exp(logstds), 10.0).T

    n_block = 131072
    while batch % n_block:
        n_block //= 2
    grid = batch // n_block

    const = lambda shape: pl.BlockSpec(shape, lambda i: (0, 0))
    out_t = jax.ShapeDtypeStruct((action_size, batch), jnp.float32)
    means_t, stds_t = pl.pallas_call(
        _actor_t_kernel,
        out_shape=(out_t, out_t),
        grid=(grid,),
        in_specs=[
            pl.BlockSpec((state_size, n_block), lambda i: (0, i)),
            const(w1t.shape), const(b1t.shape),
            const(w2t.shape), const(b2t.shape),
            const(w3t.shape), const(b3t.shape),
            const(st.shape),
        ],
        out_specs=(
            pl.BlockSpec((action_size, n_block), lambda i: (0, i)),
            pl.BlockSpec((action_size, n_block), lambda i: (0, i)),
        ),
        compiler_params=pltpu.CompilerParams(
            dimension_semantics=("parallel",)),
    )(xt, w1t, b1t, w2t, b2t, w3t, b3t, st)

    return means_t.T, stds_t.T
